# baseline (speedup 1.0000x reference)
import functools

import jax
import jax.numpy as jnp
from jax import lax
from jax.experimental import pallas as pl
from jax.experimental.pallas import tpu as pltpu
from jax.experimental.pallas import tpu_sc as plsc

N_CLS = 4096
N_CTX = 16
MAX_NAME = 16
VOCAB_ROWS = 30522
D = 768
MAX_LEN = 1 + N_CTX + MAX_NAME + 1
HEAD = 1 + N_CTX
TAIL = MAX_NAME + 1

NC = 2
NS = 16
NW = NC * NS
PER_TILE = N_CLS // NW
CHUNK = 2
NBUF = 2
STEPS = PER_TILE // (CHUNK * NBUF)


def _body(aug_hbm, ctx_hbm, ct_hbm, lens_hbm, par_hbm,
          out_hbm, mask_hbm,
          buf, ct_v, lens_v, mask_v, par_v,
          idx00, idx01, idx10, idx11, idx_cls,
          gsem0, gsem1, osem0, osem1):
    idx_refs = ((idx00, idx01), (idx10, idx11))
    gsems = (gsem0, gsem1)
    osems = (osem0, osem1)

    wid = lax.axis_index("s") * NC + lax.axis_index("c")
    base = wid * PER_TILE
    iota = lax.broadcasted_iota(jnp.int32, (16,), 0)

    pltpu.sync_copy(par_hbm, par_v)
    pltpu.sync_copy(ct_hbm.at[pl.ds(base, PER_TILE)], ct_v)
    pltpu.sync_copy(lens_hbm.at[pl.ds(base, PER_TILE)], lens_v)
    cls_v = plsc.load_gather(par_v, [iota * 0])
    sep_v = plsc.load_gather(par_v, [iota * 0 + 1])
    plsc.store_scatter(idx_cls, [iota], cls_v, mask=iota < 8)
    for b in range(NBUF):
        for c in range(CHUNK):
            pltpu.async_copy(aug_hbm.at[idx_cls],
                             buf.at[b, c, pl.ds(0, 8)], gsems[b]).wait()
            pltpu.sync_copy(ctx_hbm, buf.at[b, c, pl.ds(1, N_CTX)])

    def step(s, carry):
        for b in range(NBUF):
            g = s * NBUF + b
            c0 = base + g * CHUNK

            @pl.when(g >= NBUF)
            def _():
                pltpu.make_async_copy(buf.at[b], out_hbm.at[pl.ds(0, CHUNK)],
                                      osems[b]).wait()

            descs = []
            for c in range(CHUNK):
                local = g * CHUNK + c
                lsp = jnp.full((16,), local, jnp.int32)
                tok = plsc.load_gather(ct_v, [lsp, iota])
                lenv = plsc.load_gather(lens_v, [lsp])
                pad_v = jnp.full((16,), VOCAB_ROWS, jnp.int32)
                idx16 = jnp.where(iota < lenv, tok,
                                  jnp.where(iota == lenv, sep_v, pad_v))
                iref = idx_refs[b][c]
                iref[pl.ds(0, 16)] = idx16
                last_v = jnp.where(lenv == MAX_NAME, sep_v, pad_v)
                plsc.store_scatter(iref, [iota * 0 + MAX_NAME], last_v,
                                   mask=iota == 0)

                cur = lenv + 18
                plsc.store_scatter(mask_v, [lsp, iota],
                                   jnp.full((16,), 1, jnp.int32))
                plsc.store_scatter(mask_v, [lsp, iota + 16],
                                   (iota + 16 < cur).astype(jnp.int32))
                plsc.store_scatter(mask_v, [lsp, iota + 32],
                                   (iota + 32 < cur).astype(jnp.int32),
                                   mask=iota < 2)

                descs.append(pltpu.async_copy(
                    aug_hbm.at[iref], buf.at[b, c, pl.ds(HEAD, TAIL)],
                    gsems[b]))
            for dsc in descs:
                dsc.wait()
            pltpu.async_copy(buf.at[b], out_hbm.at[pl.ds(c0, CHUNK)], osems[b])
        return carry

    lax.fori_loop(0, STEPS, step, 0)

    for b in range(NBUF):
        pltpu.make_async_copy(buf.at[b], out_hbm.at[pl.ds(0, CHUNK)],
                              osems[b]).wait()
    pltpu.sync_copy(mask_v, mask_hbm.at[pl.ds(base, PER_TILE)])


@functools.partial(jax.jit, static_argnums=())
def _sc_call(aug, ctx, class_tokens, lens, par):
    mesh = plsc.VectorSubcoreMesh(core_axis_name="c", subcore_axis_name="s")
    f = pl.kernel(
        _body,
        mesh=mesh,
        compiler_params=pltpu.CompilerParams(use_tc_tiling_on_sc=False,
                                             needs_layout_passes=False),
        out_type=(
            jax.ShapeDtypeStruct((N_CLS, MAX_LEN, D), jnp.float32),
            jax.ShapeDtypeStruct((N_CLS, MAX_LEN), jnp.int32),
        ),
        scratch_types=[
            pltpu.VMEM((NBUF, CHUNK, MAX_LEN, D), jnp.float32),
            pltpu.VMEM((PER_TILE, MAX_NAME), jnp.int32),
            pltpu.VMEM((PER_TILE,), jnp.int32),
            pltpu.VMEM((PER_TILE, MAX_LEN), jnp.int32),
            pltpu.VMEM((8,), jnp.int32),
            pltpu.VMEM((TAIL,), jnp.int32),
            pltpu.VMEM((TAIL,), jnp.int32),
            pltpu.VMEM((TAIL,), jnp.int32),
            pltpu.VMEM((TAIL,), jnp.int32),
            pltpu.VMEM((8,), jnp.int32),
            pltpu.SemaphoreType.DMA,
            pltpu.SemaphoreType.DMA,
            pltpu.SemaphoreType.DMA,
            pltpu.SemaphoreType.DMA,
        ],
    )
    return f(aug, ctx, class_tokens, lens, par)


def kernel(table, ctx, class_tokens, lens, cls_id, sep_id):
    aug = jnp.concatenate([table, jnp.zeros((1, D), table.dtype)], axis=0)
    par = (jnp.zeros((8,), jnp.int32)
           .at[0].set(jnp.asarray(cls_id, jnp.int32))
           .at[1].set(jnp.asarray(sep_id, jnp.int32)))
    out_embeds, out_mask = _sc_call(aug, ctx, class_tokens, lens, par)
    return out_embeds, out_mask


# speedup vs baseline: 1.0698x; 1.0698x over previous
import functools

import jax
import jax.numpy as jnp
from jax import lax
from jax.experimental import pallas as pl
from jax.experimental.pallas import tpu as pltpu
from jax.experimental.pallas import tpu_sc as plsc

N_CLS = 4096
N_CTX = 16
MAX_NAME = 16
D = 768
MAX_LEN = 1 + N_CTX + MAX_NAME + 1
HEAD = 1 + N_CTX
TAIL = MAX_NAME + 1

NC = 2
NS = 16
NW = NC * NS
PER_TILE = N_CLS // NW
K = 4
NBUF = 2
STEPS = PER_TILE // (K * NBUF)


def _sc_body(table_hbm, ct_hbm, lens_hbm, par_hbm,
             t_hbm,
             stag0, stag1, gidx0, gidx1, ct_v, lens_v, par_v,
             gsem0, gsem1, osem0, osem1):
    stags = (stag0, stag1)
    gidxs = (gidx0, gidx1)
    gsems = (gsem0, gsem1)
    osems = (osem0, osem1)

    wid = lax.axis_index("s") * NC + lax.axis_index("c")
    base = wid * PER_TILE
    iota = lax.broadcasted_iota(jnp.int32, (16,), 0)

    pltpu.sync_copy(par_hbm, par_v)
    pltpu.sync_copy(ct_hbm.at[pl.ds(base, PER_TILE)], ct_v)
    pltpu.sync_copy(lens_hbm.at[pl.ds(base, PER_TILE)], lens_v)
    sep_v = plsc.load_gather(par_v, [iota * 0 + 1])

    def fill_idx(b, g):
        for c in range(K):
            local = g * K + c
            lsp = jnp.full((16,), local, jnp.int32)
            tok = plsc.load_gather(ct_v, [lsp, iota])
            lenv = plsc.load_gather(lens_v, [lsp])
            idx16 = jnp.where(iota < lenv, tok, sep_v)
            plsc.store_scatter(gidxs[b], [iota * 0 + (c * TAIL) + iota], idx16)
            plsc.store_scatter(gidxs[b], [iota * 0 + (c * TAIL + 16)], sep_v,
                               mask=iota == 0)

    def step(s, carry):
        for b in range(NBUF):
            g = s * NBUF + b

            @pl.when(g >= NBUF)
            def _():
                pltpu.make_async_copy(
                    stags[b], t_hbm.at[pl.ds(0, K * TAIL)], osems[b]).wait()

            fill_idx(b, g)
            pltpu.async_copy(table_hbm.at[gidxs[b]], stags[b], gsems[b])
        for b in range(NBUF):
            g = s * NBUF + b
            r0 = (base + g * K) * TAIL
            pltpu.make_async_copy(
                table_hbm.at[gidxs[b]], stags[b], gsems[b]).wait()
            pltpu.async_copy(stags[b], t_hbm.at[pl.ds(r0, K * TAIL)], osems[b])
        return carry

    lax.fori_loop(0, STEPS, step, 0)
    for b in range(NBUF):
        pltpu.make_async_copy(stags[b], t_hbm.at[pl.ds(0, K * TAIL)],
                              osems[b]).wait()


def _sc_gather(table, class_tokens, lens, par):
    mesh = plsc.VectorSubcoreMesh(core_axis_name="c", subcore_axis_name="s")
    f = pl.kernel(
        _sc_body,
        mesh=mesh,
        compiler_params=pltpu.CompilerParams(use_tc_tiling_on_sc=False,
                                             needs_layout_passes=False),
        out_type=jax.ShapeDtypeStruct((N_CLS * TAIL, D), jnp.float32),
        scratch_types=[
            pltpu.VMEM((K * TAIL, D), jnp.float32),
            pltpu.VMEM((K * TAIL, D), jnp.float32),
            pltpu.VMEM((K * TAIL,), jnp.int32),
            pltpu.VMEM((K * TAIL,), jnp.int32),
            pltpu.VMEM((PER_TILE, MAX_NAME), jnp.int32),
            pltpu.VMEM((PER_TILE,), jnp.int32),
            pltpu.VMEM((8,), jnp.int32),
            pltpu.SemaphoreType.DMA,
            pltpu.SemaphoreType.DMA,
            pltpu.SemaphoreType.DMA,
            pltpu.SemaphoreType.DMA,
        ],
    )
    return f(table, class_tokens, lens, par)


BC = 64


def _tc_body(t_ref, base_ref, lens_ref, out_ref, mask_ref):
    lenb = lens_ref[...]
    s_iota = lax.broadcasted_iota(jnp.int32, (BC, TAIL, 1), 1)
    tail = jnp.where(s_iota <= lenb[:, :, None], t_ref[...], 0.0)
    head = jnp.broadcast_to(base_ref[...][None], (BC, HEAD, D))
    out_ref[:, pl.ds(0, HEAD), :] = head
    out_ref[:, pl.ds(HEAD, TAIL), :] = tail
    p_iota = lax.broadcasted_iota(jnp.int32, (BC, MAX_LEN), 1)
    mask_ref[...] = (p_iota < 18 + lenb).astype(jnp.int32)


def _tc_assemble(t, base, lens2):
    return pl.pallas_call(
        _tc_body,
        grid=(N_CLS // BC,),
        in_specs=[
            pl.BlockSpec((BC, TAIL, D), lambda i: (i, 0, 0)),
            pl.BlockSpec((HEAD, D), lambda i: (0, 0)),
            pl.BlockSpec((BC, 1), lambda i: (i, 0)),
        ],
        out_specs=[
            pl.BlockSpec((BC, MAX_LEN, D), lambda i: (i, 0, 0)),
            pl.BlockSpec((BC, MAX_LEN), lambda i: (i, 0)),
        ],
        out_shape=[
            jax.ShapeDtypeStruct((N_CLS, MAX_LEN, D), jnp.float32),
            jax.ShapeDtypeStruct((N_CLS, MAX_LEN), jnp.int32),
        ],
    )(t, base, lens2)


def kernel(table, ctx, class_tokens, lens, cls_id, sep_id):
    par = (jnp.zeros((8,), jnp.int32)
           .at[0].set(jnp.asarray(cls_id, jnp.int32))
           .at[1].set(jnp.asarray(sep_id, jnp.int32)))
    t = _sc_gather(table, class_tokens, lens, par)
    t = t.reshape(N_CLS, TAIL, D)
    base = jnp.concatenate([table[cls_id][None, :], ctx], axis=0)
    out_embeds, out_mask = _tc_assemble(t, base, lens[:, None])
    return out_embeds, out_mask
